# baseline (speedup 1.0000x reference)
import jax
import jax.numpy as jnp
from jax.experimental import pallas as pl
from jax.experimental.pallas import tpu as pltpu


_CHUNK = 1024
_DEPTH = 4


def _add_kernel(x_hbm, e_hbm, o_hbm, xbuf, ebuf, obuf, xsem, esem, osem):
    n_chunks = x_hbm.shape[0]
    n_e = ebuf.shape[0]

    for j in range(n_e):
        pltpu.make_async_copy(e_hbm.at[j], ebuf.at[j], esem.at[j]).start()
    for c in range(_DEPTH):
        pltpu.make_async_copy(x_hbm.at[c], xbuf.at[c], xsem.at[c]).start()

    for c in range(n_chunks):
        slot = c % _DEPTH
        if c < n_e:
            pltpu.make_async_copy(e_hbm.at[c], ebuf.at[c], esem.at[c]).wait()
        pltpu.make_async_copy(x_hbm.at[c], xbuf.at[slot], xsem.at[slot]).wait()
        if c >= _DEPTH:
            pltpu.make_async_copy(
                obuf.at[slot], o_hbm.at[c - _DEPTH], osem.at[slot]).wait()
        obuf[slot] = xbuf[slot] + ebuf[c % n_e]
        pltpu.make_async_copy(obuf.at[slot], o_hbm.at[c], osem.at[slot]).start()
        if c + _DEPTH < n_chunks:
            pltpu.make_async_copy(
                x_hbm.at[c + _DEPTH], xbuf.at[slot], xsem.at[slot]).start()

    for c in range(n_chunks - _DEPTH, n_chunks):
        slot = c % _DEPTH
        pltpu.make_async_copy(obuf.at[slot], o_hbm.at[c], osem.at[slot]).wait()


def kernel(x, embed_weight):
    B, S, D = x.shape
    n_chunks = (B * S) // _CHUNK
    xf = x.reshape(n_chunks, _CHUNK, D)
    ef = embed_weight.reshape(embed_weight.shape[0] // _CHUNK, _CHUNK, D)
    n_e = S // _CHUNK
    out = pl.pallas_call(
        _add_kernel,
        in_specs=[
            pl.BlockSpec(memory_space=pl.ANY),
            pl.BlockSpec(memory_space=pl.ANY),
        ],
        out_specs=pl.BlockSpec(memory_space=pl.ANY),
        out_shape=jax.ShapeDtypeStruct((n_chunks, _CHUNK, D), x.dtype),
        scratch_shapes=[
            pltpu.VMEM((_DEPTH, _CHUNK, D), x.dtype),
            pltpu.VMEM((n_e, _CHUNK, D), x.dtype),
            pltpu.VMEM((_DEPTH, _CHUNK, D), x.dtype),
            pltpu.SemaphoreType.DMA((_DEPTH,)),
            pltpu.SemaphoreType.DMA((n_e,)),
            pltpu.SemaphoreType.DMA((_DEPTH,)),
        ],
    )(xf, ef)
    return out.reshape(B, S, D)


# speedup vs baseline: 1.0121x; 1.0121x over previous
import jax
import jax.numpy as jnp
from jax.experimental import pallas as pl
from jax.experimental.pallas import tpu as pltpu


_CHUNK = 1024
_DEPTH = 5


def _add_kernel(x_hbm, e_hbm, o_hbm, xbuf, ebuf, obuf, xsem, esem, osem):
    n_chunks = x_hbm.shape[0]
    n_e = ebuf.shape[0]

    for j in range(n_e):
        pltpu.make_async_copy(e_hbm.at[j], ebuf.at[j], esem.at[j]).start()
    for c in range(_DEPTH):
        pltpu.make_async_copy(x_hbm.at[c], xbuf.at[c], xsem.at[c]).start()
    for j in range(n_e):
        pltpu.make_async_copy(e_hbm.at[j], ebuf.at[j], esem.at[j]).wait()

    for c in range(n_chunks):
        slot = c % _DEPTH
        pltpu.make_async_copy(x_hbm.at[c], xbuf.at[slot], xsem.at[slot]).wait()
        if c >= _DEPTH:
            pltpu.make_async_copy(
                obuf.at[slot], o_hbm.at[c - _DEPTH], osem.at[slot]).wait()
        obuf[slot] = xbuf[slot] + ebuf[c % n_e]
        pltpu.make_async_copy(obuf.at[slot], o_hbm.at[c], osem.at[slot]).start()
        if c + _DEPTH < n_chunks:
            pltpu.make_async_copy(
                x_hbm.at[c + _DEPTH], xbuf.at[slot], xsem.at[slot]).start()

    for c in range(n_chunks - _DEPTH, n_chunks):
        slot = c % _DEPTH
        pltpu.make_async_copy(obuf.at[slot], o_hbm.at[c], osem.at[slot]).wait()


def kernel(x, embed_weight):
    B, S, D = x.shape
    n_chunks = (B * S) // _CHUNK
    xf = x.reshape(n_chunks, _CHUNK, D)
    ef = embed_weight.reshape(embed_weight.shape[0] // _CHUNK, _CHUNK, D)
    n_e = S // _CHUNK
    out = pl.pallas_call(
        _add_kernel,
        in_specs=[
            pl.BlockSpec(memory_space=pl.ANY),
            pl.BlockSpec(memory_space=pl.ANY),
        ],
        out_specs=pl.BlockSpec(memory_space=pl.ANY),
        out_shape=jax.ShapeDtypeStruct((n_chunks, _CHUNK, D), x.dtype),
        scratch_shapes=[
            pltpu.VMEM((_DEPTH, _CHUNK, D), x.dtype),
            pltpu.VMEM((n_e, _CHUNK, D), x.dtype),
            pltpu.VMEM((_DEPTH, _CHUNK, D), x.dtype),
            pltpu.SemaphoreType.DMA((_DEPTH,)),
            pltpu.SemaphoreType.DMA((n_e,)),
            pltpu.SemaphoreType.DMA((_DEPTH,)),
        ],
    )(xf, ef)
    return out.reshape(B, S, D)
